# baseline (speedup 1.0000x reference)
import jax
import jax.numpy as jnp
from jax import lax
from jax.experimental import pallas as pl
from jax.experimental.pallas import tpu as pltpu
from jax.experimental.pallas import tpu_sc as plsc

VOCAB = 1000000
EMB = 64
SEQ = 200
BATCH = 1024

NC = 2
NS = 16
L = 16
NW = NC * NS
ROWS = SEQ * BATCH
RPW = ROWS // NW
CHUNK = 800
NCHUNK = RPW // CHUNK
G = 100
NG = CHUNK // G
BPC = CHUNK // SEQ


def _emb_body(idx_hbm, table_hbm, pos_hbm, out_hbm, idx_v, rows_v, pos_v, sem):
    wid = lax.axis_index("s") * NC + lax.axis_index("c")
    base = wid * RPW
    pltpu.sync_copy(pos_hbm, pos_v)
    for c in range(NCHUNK):
        cbase = pl.multiple_of(base + c * CHUNK, CHUNK)
        pltpu.sync_copy(idx_hbm.at[pl.ds(pl.multiple_of(cbase // G, NG), NG)], idx_v)
        copies = [
            pltpu.async_copy(
                table_hbm.at[idx_v.at[j]], rows_v.at[pl.ds(j * G, G)], sem
            )
            for j in range(NG)
        ]
        for cp in copies:
            cp.wait()

        def body(t, carry):
            for j in range(EMB // L):
                p = pos_v[t, pl.ds(j * L, L)]
                for b in range(BPC):
                    r = b * SEQ + t
                    rows_v[r, pl.ds(j * L, L)] = rows_v[r, pl.ds(j * L, L)] + p
            return carry

        lax.fori_loop(0, SEQ, body, 0)
        pltpu.sync_copy(rows_v, out_hbm.at[pl.ds(cbase, CHUNK)])


def kernel(sentence, word_table, pos_table):
    idx = jnp.transpose(sentence, (1, 0)).reshape(ROWS // G, G)
    pos = lax.slice_in_dim(pos_table, 1, SEQ + 1, axis=0)
    mesh = plsc.VectorSubcoreMesh(core_axis_name="c", subcore_axis_name="s")
    out = pl.kernel(
        _emb_body,
        out_type=jax.ShapeDtypeStruct((ROWS, EMB), jnp.float32),
        mesh=mesh,
        compiler_params=pltpu.CompilerParams(use_tc_tiling_on_sc=False),
        scratch_types=[
            pltpu.VMEM((NG, G), jnp.int32),
            pltpu.VMEM((CHUNK, EMB), jnp.float32),
            pltpu.VMEM((SEQ, EMB), jnp.float32),
            pltpu.SemaphoreType.DMA,
        ],
    )(idx, word_table, pos)
    return out.reshape(BATCH, SEQ, EMB)
